# baseline (speedup 1.0000x reference)
import functools

import jax
import jax.numpy as jnp
from jax import lax
from jax.experimental import pallas as pl
from jax.experimental.pallas import tpu as pltpu
from jax.experimental.pallas import tpu_sc as plsc

N_BUSES = 10000
N_LINES = 160000
N_BATCH = 64
LANES = 16
CHUNK = 8000
N_CHUNKS = N_LINES // CHUNK
ROWS_PER_TILE = 2
UNROLL = 4


def _sc_kernel(angles_hbm, fi_hbm, ti_hbm, rl_hbm, ir_hbm,
               ang2_out, flows_out,
               ang_a, ang_b, ang2_a, ang2_b,
               fi0, ti0, r0, fi1, ti1, r1,
               fba0, fbb0, fba1, fbb1,
               sin0, sin1, sout0, sout1):
    c = lax.axis_index("c")
    s = lax.axis_index("s")
    wid = s * 2 + c
    row_a = wid * ROWS_PER_TILE
    row_b = row_a + 1

    IN = ((fi0, ti0, r0, sin0), (fi1, ti1, r1, sin1))
    OUT = ((fba0, fbb0, sout0), (fba1, fbb1, sout1))

    def start_in(b, base, r_hbm):
        fib, tib, rb, sem = IN[b]
        pltpu.async_copy(fi_hbm.at[pl.ds(base, CHUNK)], fib, sem)
        pltpu.async_copy(ti_hbm.at[pl.ds(base, CHUNK)], tib, sem)
        pltpu.async_copy(r_hbm.at[pl.ds(base, CHUNK)], rb, sem)

    def wait_in(b):
        fib, tib, rb, sem = IN[b]
        pltpu.make_async_copy(fi_hbm.at[pl.ds(0, CHUNK)], fib, sem).wait()
        pltpu.make_async_copy(ti_hbm.at[pl.ds(0, CHUNK)], tib, sem).wait()
        pltpu.make_async_copy(rl_hbm.at[pl.ds(0, CHUNK)], rb, sem).wait()

    def start_out(b, base):
        fba, fbb, sem = OUT[b]
        pltpu.async_copy(
            fba, flows_out.at[pl.ds(row_a * N_LINES + base, CHUNK)], sem)
        pltpu.async_copy(
            fbb, flows_out.at[pl.ds(row_b * N_LINES + base, CHUNK)], sem)

    def wait_out(b):
        fba, fbb, sem = OUT[b]
        pltpu.make_async_copy(fba, flows_out.at[pl.ds(0, CHUNK)], sem).wait()
        pltpu.make_async_copy(fbb, flows_out.at[pl.ds(0, CHUNK)], sem).wait()

    start_in(0, 0, rl_hbm)
    for dst in (ang_a, ang2_a):
        pltpu.async_copy(
            angles_hbm.at[pl.ds(row_a * N_BUSES, N_BUSES)], dst, sout0)
    for dst in (ang_b, ang2_b):
        pltpu.async_copy(
            angles_hbm.at[pl.ds(row_b * N_BUSES, N_BUSES)], dst, sout0)
    for dst in (ang_a, ang2_a, ang_b, ang2_b):
        pltpu.make_async_copy(
            angles_hbm.at[pl.ds(0, N_BUSES)], dst, sout0).wait()


    @pl.loop(0, N_CHUNKS, step=2)
    def phase1(ci):
        for b in range(2):
            cur = ci + b
            wait_in(b)

            @pl.when(cur + 1 < N_CHUNKS)
            def _():
                start_in(1 - b, (cur + 1) * CHUNK, rl_hbm)

            fib, tib, rb, _sem = IN[b]

            @plsc.parallel_loop(0, CHUNK, LANES, unroll=UNROLL)
            def vec1(o):
                fidx = fib[pl.ds(o, LANES)]
                tidx = tib[pl.ds(o, LANES)]
                rl = rb[pl.ds(o, LANES)]
                nrl = -rl
                for ang, ang2 in ((ang_a, ang2_a), (ang_b, ang2_b)):
                    fa = plsc.load_gather(ang, [fidx])
                    ta = plsc.load_gather(ang, [tidx])
                    d = fa - ta
                    adj = (jnp.minimum(jnp.maximum(d, nrl), rl) - d) * 0.5
                    plsc.addupdate_scatter(ang2, [fidx], adj)
                    plsc.addupdate_scatter(ang2, [tidx], adj)

    start_in(0, 0, ir_hbm)

    @pl.loop(0, N_CHUNKS, step=2)
    def phase2(ci):
        for b in range(2):
            cur = ci + b
            wait_in(b)

            @pl.when(cur + 1 < N_CHUNKS)
            def _():
                start_in(1 - b, (cur + 1) * CHUNK, ir_hbm)

            @pl.when(cur >= 2)
            def _():
                wait_out(b)

            fib, tib, rb, _sem = IN[b]
            fba, fbb, _osem = OUT[b]

            @plsc.parallel_loop(0, CHUNK, LANES, unroll=UNROLL)
            def vec2(o):
                fidx = fib[pl.ds(o, LANES)]
                tidx = tib[pl.ds(o, LANES)]
                ir = rb[pl.ds(o, LANES)]
                for ang2, fbuf in ((ang2_a, fba), (ang2_b, fbb)):
                    fa = plsc.load_gather(ang2, [fidx])
                    ta = plsc.load_gather(ang2, [tidx])
                    fbuf[pl.ds(o, LANES)] = (fa - ta) * ir

            start_out(b, cur * CHUNK)

    wait_out(0)
    wait_out(1)
    pltpu.sync_copy(ang2_a, ang2_out.at[pl.ds(row_a * N_BUSES, N_BUSES)])
    pltpu.sync_copy(ang2_b, ang2_out.at[pl.ds(row_b * N_BUSES, N_BUSES)])


@jax.jit
def _run(angles, from_indices, to_indices, rl, inv_r):
    mesh = plsc.VectorSubcoreMesh(core_axis_name="c", subcore_axis_name="s")
    f = functools.partial(
        pl.kernel,
        mesh=mesh,
        compiler_params=pltpu.CompilerParams(needs_layout_passes=False),
        out_type=[
            jax.ShapeDtypeStruct((N_BATCH * N_BUSES,), jnp.float32),
            jax.ShapeDtypeStruct((N_BATCH * N_LINES,), jnp.float32),
        ],
        scratch_types=[
            pltpu.VMEM((N_BUSES,), jnp.float32),
            pltpu.VMEM((N_BUSES,), jnp.float32),
            pltpu.VMEM((N_BUSES,), jnp.float32),
            pltpu.VMEM((N_BUSES,), jnp.float32),
            pltpu.VMEM((CHUNK,), jnp.int32),
            pltpu.VMEM((CHUNK,), jnp.int32),
            pltpu.VMEM((CHUNK,), jnp.float32),
            pltpu.VMEM((CHUNK,), jnp.int32),
            pltpu.VMEM((CHUNK,), jnp.int32),
            pltpu.VMEM((CHUNK,), jnp.float32),
            pltpu.VMEM((CHUNK,), jnp.float32),
            pltpu.VMEM((CHUNK,), jnp.float32),
            pltpu.VMEM((CHUNK,), jnp.float32),
            pltpu.VMEM((CHUNK,), jnp.float32),
            pltpu.SemaphoreType.DMA,
            pltpu.SemaphoreType.DMA,
            pltpu.SemaphoreType.DMA,
            pltpu.SemaphoreType.DMA,
        ],
    )(_sc_kernel)
    return f(angles, from_indices, to_indices, rl, inv_r)


def kernel(x, from_indices, to_indices, reactances, limits):
    angles = x[:, N_BUSES:2 * N_BUSES].reshape(-1)
    angles2, flows2 = _run(
        angles,
        from_indices.astype(jnp.int32),
        to_indices.astype(jnp.int32),
        reactances * limits,
        1.0 / reactances,
    )
    angles2 = angles2.reshape(N_BATCH, N_BUSES)
    flows2 = flows2.reshape(N_BATCH, N_LINES)
    out = jnp.concatenate(
        [x[:, :N_BUSES], angles2, x[:, 2 * N_BUSES:]], axis=1)
    return (out, flows2)


# speedup vs baseline: 1.0333x; 1.0333x over previous
import functools

import jax
import jax.numpy as jnp
from jax import lax
from jax.experimental import pallas as pl
from jax.experimental.pallas import tpu as pltpu
from jax.experimental.pallas import tpu_sc as plsc

N_BUSES = 10000
N_LINES = 160000
N_BATCH = 64
LANES = 16
CHUNK = 8000
N_CHUNKS = N_LINES // CHUNK
ROWS_PER_TILE = 2
UNROLL = 4


def _sc_kernel(angles_hbm, fi_hbm, ti_hbm, rl_hbm, ir_hbm,
               ang2_out, flows_out,
               ang_a, ang_b, ang2_a, ang2_b,
               fi0, ti0, r0, fi1, ti1, r1,
               fba0, fbb0, fba1, fbb1,
               sin0, sin1, sout0, sout1):
    c = lax.axis_index("c")
    s = lax.axis_index("s")
    wid = s * 2 + c
    row_a = wid * ROWS_PER_TILE
    row_b = row_a + 1

    IN = ((fi0, ti0, r0, sin0), (fi1, ti1, r1, sin1))
    OUT = ((fba0, fbb0, sout0), (fba1, fbb1, sout1))

    def start_in(b, base, r_hbm):
        fib, tib, rb, sem = IN[b]
        pltpu.async_copy(fi_hbm.at[pl.ds(base, CHUNK)], fib, sem)
        pltpu.async_copy(ti_hbm.at[pl.ds(base, CHUNK)], tib, sem)
        pltpu.async_copy(r_hbm.at[pl.ds(base, CHUNK)], rb, sem)

    def wait_in(b):
        fib, tib, rb, sem = IN[b]
        pltpu.make_async_copy(fi_hbm.at[pl.ds(0, CHUNK)], fib, sem).wait()
        pltpu.make_async_copy(ti_hbm.at[pl.ds(0, CHUNK)], tib, sem).wait()
        pltpu.make_async_copy(rl_hbm.at[pl.ds(0, CHUNK)], rb, sem).wait()

    def start_out(b, base):
        fba, fbb, sem = OUT[b]
        pltpu.async_copy(
            fba, flows_out.at[pl.ds(row_a * N_LINES + base, CHUNK)], sem)
        pltpu.async_copy(
            fbb, flows_out.at[pl.ds(row_b * N_LINES + base, CHUNK)], sem)

    def wait_out(b):
        fba, fbb, sem = OUT[b]
        pltpu.make_async_copy(fba, flows_out.at[pl.ds(0, CHUNK)], sem).wait()
        pltpu.make_async_copy(fbb, flows_out.at[pl.ds(0, CHUNK)], sem).wait()

    start_in(0, 0, rl_hbm)
    for dst in (ang_a, ang2_a):
        pltpu.async_copy(
            angles_hbm.at[pl.ds(row_a * N_BUSES, N_BUSES)], dst, sout0)
    for dst in (ang_b, ang2_b):
        pltpu.async_copy(
            angles_hbm.at[pl.ds(row_b * N_BUSES, N_BUSES)], dst, sout0)
    for dst in (ang_a, ang2_a, ang_b, ang2_b):
        pltpu.make_async_copy(
            angles_hbm.at[pl.ds(0, N_BUSES)], dst, sout0).wait()


    @pl.loop(0, N_CHUNKS, step=2)
    def phase1(ci):
        for b in range(2):
            cur = ci + b
            wait_in(b)

            @pl.when(cur + 1 < N_CHUNKS)
            def _():
                start_in(1 - b, (cur + 1) * CHUNK, rl_hbm)

            fib, tib, rb, _sem = IN[b]

            @plsc.parallel_loop(0, CHUNK, LANES, unroll=UNROLL)
            def vec1(o):
                fidx = fib[pl.ds(o, LANES)]
                tidx = tib[pl.ds(o, LANES)]
                rl = rb[pl.ds(o, LANES)]
                for ang, ang2 in ((ang_a, ang2_a), (ang_b, ang2_b)):
                    fa = plsc.load_gather(ang, [fidx])
                    ta = plsc.load_gather(ang, [tidx])
                    d = fa - ta
                    over = jnp.abs(d) > rl
                    adj = (jnp.sign(d) * rl - d) * 0.5
                    plsc.addupdate_scatter(ang2, [fidx], adj, mask=over)
                    plsc.addupdate_scatter(ang2, [tidx], adj, mask=over)

    start_in(0, 0, ir_hbm)

    @pl.loop(0, N_CHUNKS, step=2)
    def phase2(ci):
        for b in range(2):
            cur = ci + b
            wait_in(b)

            @pl.when(cur + 1 < N_CHUNKS)
            def _():
                start_in(1 - b, (cur + 1) * CHUNK, ir_hbm)

            @pl.when(cur >= 2)
            def _():
                wait_out(b)

            fib, tib, rb, _sem = IN[b]
            fba, fbb, _osem = OUT[b]

            @plsc.parallel_loop(0, CHUNK, LANES, unroll=UNROLL)
            def vec2(o):
                fidx = fib[pl.ds(o, LANES)]
                tidx = tib[pl.ds(o, LANES)]
                ir = rb[pl.ds(o, LANES)]
                for ang2, fbuf in ((ang2_a, fba), (ang2_b, fbb)):
                    fa = plsc.load_gather(ang2, [fidx])
                    ta = plsc.load_gather(ang2, [tidx])
                    fbuf[pl.ds(o, LANES)] = (fa - ta) * ir

            start_out(b, cur * CHUNK)

    wait_out(0)
    wait_out(1)
    pltpu.sync_copy(ang2_a, ang2_out.at[pl.ds(row_a * N_BUSES, N_BUSES)])
    pltpu.sync_copy(ang2_b, ang2_out.at[pl.ds(row_b * N_BUSES, N_BUSES)])


@jax.jit
def _run(angles, from_indices, to_indices, rl, inv_r):
    mesh = plsc.VectorSubcoreMesh(core_axis_name="c", subcore_axis_name="s")
    f = functools.partial(
        pl.kernel,
        mesh=mesh,
        compiler_params=pltpu.CompilerParams(needs_layout_passes=False),
        out_type=[
            jax.ShapeDtypeStruct((N_BATCH * N_BUSES,), jnp.float32),
            jax.ShapeDtypeStruct((N_BATCH * N_LINES,), jnp.float32),
        ],
        scratch_types=[
            pltpu.VMEM((N_BUSES,), jnp.float32),
            pltpu.VMEM((N_BUSES,), jnp.float32),
            pltpu.VMEM((N_BUSES,), jnp.float32),
            pltpu.VMEM((N_BUSES,), jnp.float32),
            pltpu.VMEM((CHUNK,), jnp.int32),
            pltpu.VMEM((CHUNK,), jnp.int32),
            pltpu.VMEM((CHUNK,), jnp.float32),
            pltpu.VMEM((CHUNK,), jnp.int32),
            pltpu.VMEM((CHUNK,), jnp.int32),
            pltpu.VMEM((CHUNK,), jnp.float32),
            pltpu.VMEM((CHUNK,), jnp.float32),
            pltpu.VMEM((CHUNK,), jnp.float32),
            pltpu.VMEM((CHUNK,), jnp.float32),
            pltpu.VMEM((CHUNK,), jnp.float32),
            pltpu.SemaphoreType.DMA,
            pltpu.SemaphoreType.DMA,
            pltpu.SemaphoreType.DMA,
            pltpu.SemaphoreType.DMA,
        ],
    )(_sc_kernel)
    return f(angles, from_indices, to_indices, rl, inv_r)


def kernel(x, from_indices, to_indices, reactances, limits):
    angles = x[:, N_BUSES:2 * N_BUSES].reshape(-1)
    angles2, flows2 = _run(
        angles,
        from_indices.astype(jnp.int32),
        to_indices.astype(jnp.int32),
        reactances * limits,
        1.0 / reactances,
    )
    angles2 = angles2.reshape(N_BATCH, N_BUSES)
    flows2 = flows2.reshape(N_BATCH, N_LINES)
    out = jnp.concatenate(
        [x[:, :N_BUSES], angles2, x[:, 2 * N_BUSES:]], axis=1)
    return (out, flows2)


# speedup vs baseline: 1.0491x; 1.0154x over previous
import functools

import jax
import jax.numpy as jnp
from jax import lax
from jax.experimental import pallas as pl
from jax.experimental.pallas import tpu as pltpu
from jax.experimental.pallas import tpu_sc as plsc

N_BUSES = 10000
N_LINES = 160000
N_BATCH = 64
LANES = 16
CHUNK = 8000
N_CHUNKS = N_LINES // CHUNK
ROWS_PER_TILE = 2
UNROLL = 4


def _sc_kernel(angles_hbm, fi_hbm, ti_hbm, rl_hbm, ir_hbm,
               ang2_out, flows_out,
               ang_a, ang_b, ang2_a, ang2_b,
               fi0, ti0, r0, fi1, ti1, r1,
               fba0, fbb0, fba1, fbb1,
               sin0, sin1, sout0, sout1):
    c = lax.axis_index("c")
    s = lax.axis_index("s")
    wid = s * 2 + c
    row_a = wid * ROWS_PER_TILE
    row_b = row_a + 1

    IN = ((fi0, ti0, r0, sin0), (fi1, ti1, r1, sin1))
    OUT = ((fba0, fbb0, sout0), (fba1, fbb1, sout1))

    def start_in(b, base, r_hbm):
        fib, tib, rb, sem = IN[b]
        pltpu.async_copy(fi_hbm.at[pl.ds(base, CHUNK)], fib, sem)
        pltpu.async_copy(ti_hbm.at[pl.ds(base, CHUNK)], tib, sem)
        pltpu.async_copy(r_hbm.at[pl.ds(base, CHUNK)], rb, sem)

    def wait_in(b):
        fib, tib, rb, sem = IN[b]
        pltpu.make_async_copy(fi_hbm.at[pl.ds(0, CHUNK)], fib, sem).wait()
        pltpu.make_async_copy(ti_hbm.at[pl.ds(0, CHUNK)], tib, sem).wait()
        pltpu.make_async_copy(rl_hbm.at[pl.ds(0, CHUNK)], rb, sem).wait()

    def start_out(b, base):
        fba, fbb, sem = OUT[b]
        pltpu.async_copy(
            fba, flows_out.at[pl.ds(row_a * N_LINES + base, CHUNK)], sem)
        pltpu.async_copy(
            fbb, flows_out.at[pl.ds(row_b * N_LINES + base, CHUNK)], sem)

    def wait_out(b):
        fba, fbb, sem = OUT[b]
        pltpu.make_async_copy(fba, flows_out.at[pl.ds(0, CHUNK)], sem).wait()
        pltpu.make_async_copy(fbb, flows_out.at[pl.ds(0, CHUNK)], sem).wait()

    start_in(0, 0, rl_hbm)
    for dst in (ang_a, ang2_a):
        pltpu.async_copy(
            angles_hbm.at[pl.ds(row_a * N_BUSES, N_BUSES)], dst, sout0)
    for dst in (ang_b, ang2_b):
        pltpu.async_copy(
            angles_hbm.at[pl.ds(row_b * N_BUSES, N_BUSES)], dst, sout0)
    for dst in (ang_a, ang2_a, ang_b, ang2_b):
        pltpu.make_async_copy(
            angles_hbm.at[pl.ds(0, N_BUSES)], dst, sout0).wait()


    @pl.loop(0, N_CHUNKS, step=2)
    def phase1(ci):
        for b in range(2):
            cur = ci + b
            wait_in(b)

            @pl.when(cur + 1 < N_CHUNKS)
            def _():
                start_in(1 - b, (cur + 1) * CHUNK, rl_hbm)

            @pl.when(cur + 1 == N_CHUNKS)
            def _():
                start_in(1 - b, 0, ir_hbm)

            fib, tib, rb, _sem = IN[b]

            @plsc.parallel_loop(0, CHUNK, LANES, unroll=UNROLL)
            def vec1(o):
                fidx = fib[pl.ds(o, LANES)]
                tidx = tib[pl.ds(o, LANES)]
                rl = rb[pl.ds(o, LANES)]
                for ang, ang2 in ((ang_a, ang2_a), (ang_b, ang2_b)):
                    fa = plsc.load_gather(ang, [fidx])
                    ta = plsc.load_gather(ang, [tidx])
                    d = fa - ta
                    over = jnp.abs(d) > rl
                    adj = (jnp.sign(d) * rl - d) * 0.5
                    plsc.addupdate_scatter(ang2, [fidx], adj, mask=over)
                    plsc.addupdate_scatter(ang2, [tidx], adj, mask=over)

    @pl.loop(0, N_CHUNKS, step=2)
    def phase2(ci):
        for b in range(2):
            cur = ci + b
            wait_in(b)

            @pl.when(cur + 1 < N_CHUNKS)
            def _():
                start_in(1 - b, (cur + 1) * CHUNK, ir_hbm)

            @pl.when(cur >= 2)
            def _():
                wait_out(b)

            fib, tib, rb, _sem = IN[b]
            fba, fbb, _osem = OUT[b]

            @plsc.parallel_loop(0, CHUNK, LANES, unroll=UNROLL)
            def vec2(o):
                fidx = fib[pl.ds(o, LANES)]
                tidx = tib[pl.ds(o, LANES)]
                ir = rb[pl.ds(o, LANES)]
                for ang2, fbuf in ((ang2_a, fba), (ang2_b, fbb)):
                    fa = plsc.load_gather(ang2, [fidx])
                    ta = plsc.load_gather(ang2, [tidx])
                    fbuf[pl.ds(o, LANES)] = (fa - ta) * ir

            start_out(b, cur * CHUNK)

    wait_out(0)
    wait_out(1)
    pltpu.sync_copy(ang2_a, ang2_out.at[pl.ds(row_a * N_BUSES, N_BUSES)])
    pltpu.sync_copy(ang2_b, ang2_out.at[pl.ds(row_b * N_BUSES, N_BUSES)])


@jax.jit
def _run(angles, from_indices, to_indices, rl, inv_r):
    mesh = plsc.VectorSubcoreMesh(core_axis_name="c", subcore_axis_name="s")
    f = functools.partial(
        pl.kernel,
        mesh=mesh,
        compiler_params=pltpu.CompilerParams(needs_layout_passes=False),
        out_type=[
            jax.ShapeDtypeStruct((N_BATCH * N_BUSES,), jnp.float32),
            jax.ShapeDtypeStruct((N_BATCH * N_LINES,), jnp.float32),
        ],
        scratch_types=[
            pltpu.VMEM((N_BUSES,), jnp.float32),
            pltpu.VMEM((N_BUSES,), jnp.float32),
            pltpu.VMEM((N_BUSES,), jnp.float32),
            pltpu.VMEM((N_BUSES,), jnp.float32),
            pltpu.VMEM((CHUNK,), jnp.int32),
            pltpu.VMEM((CHUNK,), jnp.int32),
            pltpu.VMEM((CHUNK,), jnp.float32),
            pltpu.VMEM((CHUNK,), jnp.int32),
            pltpu.VMEM((CHUNK,), jnp.int32),
            pltpu.VMEM((CHUNK,), jnp.float32),
            pltpu.VMEM((CHUNK,), jnp.float32),
            pltpu.VMEM((CHUNK,), jnp.float32),
            pltpu.VMEM((CHUNK,), jnp.float32),
            pltpu.VMEM((CHUNK,), jnp.float32),
            pltpu.SemaphoreType.DMA,
            pltpu.SemaphoreType.DMA,
            pltpu.SemaphoreType.DMA,
            pltpu.SemaphoreType.DMA,
        ],
    )(_sc_kernel)
    return f(angles, from_indices, to_indices, rl, inv_r)


def kernel(x, from_indices, to_indices, reactances, limits):
    angles = x[:, N_BUSES:2 * N_BUSES].reshape(-1)
    angles2, flows2 = _run(
        angles,
        from_indices.astype(jnp.int32),
        to_indices.astype(jnp.int32),
        reactances * limits,
        1.0 / reactances,
    )
    angles2 = angles2.reshape(N_BATCH, N_BUSES)
    flows2 = flows2.reshape(N_BATCH, N_LINES)
    out = jnp.concatenate(
        [x[:, :N_BUSES], angles2, x[:, 2 * N_BUSES:]], axis=1)
    return (out, flows2)


# speedup vs baseline: 1.1030x; 1.0513x over previous
import functools

import jax
import jax.numpy as jnp
from jax import lax
from jax.experimental import pallas as pl
from jax.experimental.pallas import tpu as pltpu
from jax.experimental.pallas import tpu_sc as plsc

N_BUSES = 10000
N_LINES = 160000
N_BATCH = 64
LANES = 16
CHUNK = 8000
N_CHUNKS = N_LINES // CHUNK
ROWS_PER_TILE = 2
UNROLL = 4
IDX_BITS = 14
IDX_MASK = (1 << IDX_BITS) - 1


def _sc_kernel(angles_hbm, pk_hbm, rl_hbm, ir_hbm,
               ang2_out, flows_out,
               ang_a, ang_b, ang2_a, ang2_b,
               pk0, r0, pk1, r1,
               fba0, fbb0, fba1, fbb1,
               sin0, sin1, sout0, sout1):
    c = lax.axis_index("c")
    s = lax.axis_index("s")
    wid = s * 2 + c
    row_a = wid * ROWS_PER_TILE
    row_b = row_a + 1

    IN = ((pk0, r0, sin0), (pk1, r1, sin1))
    OUT = ((fba0, fbb0, sout0), (fba1, fbb1, sout1))

    def start_in(b, base, r_hbm):
        pkb, rb, sem = IN[b]
        pltpu.async_copy(pk_hbm.at[pl.ds(base, CHUNK)], pkb, sem)
        pltpu.async_copy(r_hbm.at[pl.ds(base, CHUNK)], rb, sem)

    def wait_in(b):
        pkb, rb, sem = IN[b]
        pltpu.make_async_copy(pk_hbm.at[pl.ds(0, CHUNK)], pkb, sem).wait()
        pltpu.make_async_copy(rl_hbm.at[pl.ds(0, CHUNK)], rb, sem).wait()

    def start_out(b, base):
        fba, fbb, sem = OUT[b]
        pltpu.async_copy(
            fba, flows_out.at[pl.ds(row_a * N_LINES + base, CHUNK)], sem)
        pltpu.async_copy(
            fbb, flows_out.at[pl.ds(row_b * N_LINES + base, CHUNK)], sem)

    def wait_out(b):
        fba, fbb, sem = OUT[b]
        pltpu.make_async_copy(fba, flows_out.at[pl.ds(0, CHUNK)], sem).wait()
        pltpu.make_async_copy(fbb, flows_out.at[pl.ds(0, CHUNK)], sem).wait()

    start_in(0, 0, rl_hbm)
    for dst in (ang_a, ang2_a):
        pltpu.async_copy(
            angles_hbm.at[pl.ds(row_a * N_BUSES, N_BUSES)], dst, sout0)
    for dst in (ang_b, ang2_b):
        pltpu.async_copy(
            angles_hbm.at[pl.ds(row_b * N_BUSES, N_BUSES)], dst, sout0)
    for dst in (ang_a, ang2_a, ang_b, ang2_b):
        pltpu.make_async_copy(
            angles_hbm.at[pl.ds(0, N_BUSES)], dst, sout0).wait()

    @pl.loop(0, N_CHUNKS, step=2)
    def phase1(ci):
        for b in range(2):
            cur = ci + b
            wait_in(b)

            @pl.when(cur + 1 < N_CHUNKS)
            def _():
                start_in(1 - b, (cur + 1) * CHUNK, rl_hbm)

            @pl.when(cur + 1 == N_CHUNKS)
            def _():
                start_in(1 - b, 0, ir_hbm)

            pkb, rb, _sem = IN[b]

            @plsc.parallel_loop(0, CHUNK, LANES, unroll=UNROLL)
            def vec1(o):
                pk = pkb[pl.ds(o, LANES)]
                fidx = pk & IDX_MASK
                tidx = lax.shift_right_logical(pk, IDX_BITS)
                rl = rb[pl.ds(o, LANES)]
                for ang, ang2 in ((ang_a, ang2_a), (ang_b, ang2_b)):
                    fa = plsc.load_gather(ang, [fidx])
                    ta = plsc.load_gather(ang, [tidx])
                    d = fa - ta
                    over = jnp.abs(d) > rl
                    adj = (jnp.sign(d) * rl - d) * 0.5
                    plsc.addupdate_scatter(ang2, [fidx], adj, mask=over)
                    plsc.addupdate_scatter(ang2, [tidx], adj, mask=over)

    @pl.loop(0, N_CHUNKS, step=2)
    def phase2(ci):
        for b in range(2):
            cur = ci + b
            wait_in(b)

            @pl.when(cur + 1 < N_CHUNKS)
            def _():
                start_in(1 - b, (cur + 1) * CHUNK, ir_hbm)

            @pl.when(cur >= 2)
            def _():
                wait_out(b)

            pkb, rb, _sem = IN[b]
            fba, fbb, _osem = OUT[b]

            @plsc.parallel_loop(0, CHUNK, LANES, unroll=UNROLL)
            def vec2(o):
                pk = pkb[pl.ds(o, LANES)]
                fidx = pk & IDX_MASK
                tidx = lax.shift_right_logical(pk, IDX_BITS)
                ir = rb[pl.ds(o, LANES)]
                for ang2, fbuf in ((ang2_a, fba), (ang2_b, fbb)):
                    fa = plsc.load_gather(ang2, [fidx])
                    ta = plsc.load_gather(ang2, [tidx])
                    fbuf[pl.ds(o, LANES)] = (fa - ta) * ir

            start_out(b, cur * CHUNK)

    wait_out(0)
    wait_out(1)
    pltpu.sync_copy(ang2_a, ang2_out.at[pl.ds(row_a * N_BUSES, N_BUSES)])
    pltpu.sync_copy(ang2_b, ang2_out.at[pl.ds(row_b * N_BUSES, N_BUSES)])


@jax.jit
def _run(angles, packed_idx, rl, inv_r):
    mesh = plsc.VectorSubcoreMesh(core_axis_name="c", subcore_axis_name="s")
    f = functools.partial(
        pl.kernel,
        mesh=mesh,
        compiler_params=pltpu.CompilerParams(needs_layout_passes=False),
        out_type=[
            jax.ShapeDtypeStruct((N_BATCH * N_BUSES,), jnp.float32),
            jax.ShapeDtypeStruct((N_BATCH * N_LINES,), jnp.float32),
        ],
        scratch_types=[
            pltpu.VMEM((N_BUSES,), jnp.float32),
            pltpu.VMEM((N_BUSES,), jnp.float32),
            pltpu.VMEM((N_BUSES,), jnp.float32),
            pltpu.VMEM((N_BUSES,), jnp.float32),
            pltpu.VMEM((CHUNK,), jnp.int32),
            pltpu.VMEM((CHUNK,), jnp.float32),
            pltpu.VMEM((CHUNK,), jnp.int32),
            pltpu.VMEM((CHUNK,), jnp.float32),
            pltpu.VMEM((CHUNK,), jnp.float32),
            pltpu.VMEM((CHUNK,), jnp.float32),
            pltpu.VMEM((CHUNK,), jnp.float32),
            pltpu.VMEM((CHUNK,), jnp.float32),
            pltpu.SemaphoreType.DMA,
            pltpu.SemaphoreType.DMA,
            pltpu.SemaphoreType.DMA,
            pltpu.SemaphoreType.DMA,
        ],
    )(_sc_kernel)
    return f(angles, packed_idx, rl, inv_r)


def kernel(x, from_indices, to_indices, reactances, limits):
    angles = x[:, N_BUSES:2 * N_BUSES].reshape(-1)
    fi = from_indices.astype(jnp.int32)
    ti = to_indices.astype(jnp.int32)
    packed_idx = fi | (ti << IDX_BITS)
    angles2, flows2 = _run(
        angles,
        packed_idx,
        reactances * limits,
        1.0 / reactances,
    )
    angles2 = angles2.reshape(N_BATCH, N_BUSES)
    flows2 = flows2.reshape(N_BATCH, N_LINES)
    out = jnp.concatenate(
        [x[:, :N_BUSES], angles2, x[:, 2 * N_BUSES:]], axis=1)
    return (out, flows2)


# speedup vs baseline: 1.1031x; 1.0002x over previous
import functools

import jax
import jax.numpy as jnp
from jax import lax
from jax.experimental import pallas as pl
from jax.experimental.pallas import tpu as pltpu
from jax.experimental.pallas import tpu_sc as plsc

N_BUSES = 10000
N_LINES = 160000
N_BATCH = 64
LANES = 16
CHUNK = 8000
N_CHUNKS = N_LINES // CHUNK
ROWS_PER_TILE = 2
UNROLL = 4
IDX_BITS = 14
IDX_MASK = (1 << IDX_BITS) - 1


def _sc_kernel(angles_hbm, pk_hbm, rl_hbm, ir_hbm,
               ang2_out, flows_out,
               ang_a, ang_b, ang2_a, ang2_b,
               pk0, r0, pk1, r1,
               fba0, fbb0, fba1, fbb1,
               sin0, sin1, sout0, sout1):
    c = lax.axis_index("c")
    s = lax.axis_index("s")
    wid = s * 2 + c
    row_a = wid * ROWS_PER_TILE
    row_b = row_a + 1

    IN = ((pk0, r0, sin0), (pk1, r1, sin1))
    OUT = ((fba0, fbb0, sout0), (fba1, fbb1, sout1))

    def start_in(b, base, r_hbm):
        pkb, rb, sem = IN[b]
        pltpu.async_copy(pk_hbm.at[pl.ds(base, CHUNK)], pkb, sem)
        pltpu.async_copy(r_hbm.at[pl.ds(base, CHUNK)], rb, sem)

    def wait_in(b):
        pkb, rb, sem = IN[b]
        pltpu.make_async_copy(pk_hbm.at[pl.ds(0, CHUNK)], pkb, sem).wait()
        pltpu.make_async_copy(rl_hbm.at[pl.ds(0, CHUNK)], rb, sem).wait()

    def start_out(b, base):
        fba, fbb, sem = OUT[b]
        pltpu.async_copy(
            fba, flows_out.at[pl.ds(row_a * N_LINES + base, CHUNK)], sem)
        pltpu.async_copy(
            fbb, flows_out.at[pl.ds(row_b * N_LINES + base, CHUNK)], sem)

    def wait_out(b):
        fba, fbb, sem = OUT[b]
        pltpu.make_async_copy(fba, flows_out.at[pl.ds(0, CHUNK)], sem).wait()
        pltpu.make_async_copy(fbb, flows_out.at[pl.ds(0, CHUNK)], sem).wait()

    start_in(0, 0, rl_hbm)
    for dst in (ang_a, ang2_a):
        pltpu.async_copy(
            angles_hbm.at[pl.ds(row_a * N_BUSES, N_BUSES)], dst, sout0)
    for dst in (ang_b, ang2_b):
        pltpu.async_copy(
            angles_hbm.at[pl.ds(row_b * N_BUSES, N_BUSES)], dst, sout0)
    for dst in (ang_a, ang2_a, ang_b, ang2_b):
        pltpu.make_async_copy(
            angles_hbm.at[pl.ds(0, N_BUSES)], dst, sout0).wait()

    @pl.loop(0, N_CHUNKS, step=2)
    def phase1(ci):
        for b in range(2):
            cur = ci + b
            wait_in(b)

            @pl.when(cur + 1 < N_CHUNKS)
            def _():
                start_in(1 - b, (cur + 1) * CHUNK, rl_hbm)

            @pl.when(cur + 1 == N_CHUNKS)
            def _():
                start_in(1 - b, 0, ir_hbm)

            pkb, rb, _sem = IN[b]

            @plsc.parallel_loop(0, CHUNK, LANES, unroll=UNROLL)
            def vec1(o):
                pk = pkb[pl.ds(o, LANES)]
                fidx = pk & IDX_MASK
                tidx = lax.shift_right_arithmetic(pk, IDX_BITS)
                rl = rb[pl.ds(o, LANES)]
                for ang, ang2 in ((ang_a, ang2_a), (ang_b, ang2_b)):
                    fa = plsc.load_gather(ang, [fidx])
                    ta = plsc.load_gather(ang, [tidx])
                    d = fa - ta
                    over = jnp.abs(d) > rl
                    adj = (jnp.sign(d) * rl - d) * 0.5
                    plsc.addupdate_scatter(ang2, [fidx], adj, mask=over)
                    plsc.addupdate_scatter(ang2, [tidx], adj, mask=over)

    @pl.loop(0, N_CHUNKS, step=2)
    def phase2(ci):
        for b in range(2):
            cur = ci + b
            wait_in(b)

            @pl.when(cur + 1 < N_CHUNKS)
            def _():
                start_in(1 - b, (cur + 1) * CHUNK, ir_hbm)

            @pl.when(cur >= 2)
            def _():
                wait_out(b)

            pkb, rb, _sem = IN[b]
            fba, fbb, _osem = OUT[b]

            @plsc.parallel_loop(0, CHUNK, LANES, unroll=UNROLL)
            def vec2(o):
                pk = pkb[pl.ds(o, LANES)]
                fidx = pk & IDX_MASK
                tidx = lax.shift_right_arithmetic(pk, IDX_BITS)
                ir = rb[pl.ds(o, LANES)]
                for ang2, fbuf in ((ang2_a, fba), (ang2_b, fbb)):
                    fa = plsc.load_gather(ang2, [fidx])
                    ta = plsc.load_gather(ang2, [tidx])
                    fbuf[pl.ds(o, LANES)] = (fa - ta) * ir

            start_out(b, cur * CHUNK)

    wait_out(0)
    wait_out(1)
    pltpu.sync_copy(ang2_a, ang2_out.at[pl.ds(row_a * N_BUSES, N_BUSES)])
    pltpu.sync_copy(ang2_b, ang2_out.at[pl.ds(row_b * N_BUSES, N_BUSES)])


@jax.jit
def _run(angles, packed_idx, rl, inv_r):
    mesh = plsc.VectorSubcoreMesh(core_axis_name="c", subcore_axis_name="s")
    f = functools.partial(
        pl.kernel,
        mesh=mesh,
        compiler_params=pltpu.CompilerParams(needs_layout_passes=False),
        out_type=[
            jax.ShapeDtypeStruct((N_BATCH * N_BUSES,), jnp.float32),
            jax.ShapeDtypeStruct((N_BATCH * N_LINES,), jnp.float32),
        ],
        scratch_types=[
            pltpu.VMEM((N_BUSES,), jnp.float32),
            pltpu.VMEM((N_BUSES,), jnp.float32),
            pltpu.VMEM((N_BUSES,), jnp.float32),
            pltpu.VMEM((N_BUSES,), jnp.float32),
            pltpu.VMEM((CHUNK,), jnp.int32),
            pltpu.VMEM((CHUNK,), jnp.float32),
            pltpu.VMEM((CHUNK,), jnp.int32),
            pltpu.VMEM((CHUNK,), jnp.float32),
            pltpu.VMEM((CHUNK,), jnp.float32),
            pltpu.VMEM((CHUNK,), jnp.float32),
            pltpu.VMEM((CHUNK,), jnp.float32),
            pltpu.VMEM((CHUNK,), jnp.float32),
            pltpu.SemaphoreType.DMA,
            pltpu.SemaphoreType.DMA,
            pltpu.SemaphoreType.DMA,
            pltpu.SemaphoreType.DMA,
        ],
    )(_sc_kernel)
    return f(angles, packed_idx, rl, inv_r)


def kernel(x, from_indices, to_indices, reactances, limits):
    angles = x[:, N_BUSES:2 * N_BUSES].reshape(-1)
    fi = from_indices.astype(jnp.int32)
    ti = to_indices.astype(jnp.int32)
    packed_idx = fi | (ti << IDX_BITS)
    angles2, flows2 = _run(
        angles,
        packed_idx,
        reactances * limits,
        1.0 / reactances,
    )
    angles2 = angles2.reshape(N_BATCH, N_BUSES)
    flows2 = flows2.reshape(N_BATCH, N_LINES)
    out = jnp.concatenate(
        [x[:, :N_BUSES], angles2, x[:, 2 * N_BUSES:]], axis=1)
    return (out, flows2)
